# baseline (speedup 1.0000x reference)
import functools

import jax
import jax.numpy as jnp
from jax import lax
from jax.experimental import pallas as pl
from jax.experimental.pallas import tpu as pltpu
from jax.experimental.pallas import tpu_sc as plsc

B, S, HID, MAXP = 4, 4096, 1024, 4096
EPS = 1e-12
N = B * S

N_SC = 12288
N_TC = N - N_SC
RB = 256
NB_TC = N_TC // RB

LANES = 16
VPR = HID // LANES
CH = 16
NC, NS = 2, 16
NW = NC * NS
ROWS_PER_W = N_SC // NW
NCHUNKS = ROWS_PER_W // CH
NBUF = 2
NACC = 4
STATS_OFF = CH * 2 * LANES


def _rsqrt(y):
    i = lax.bitcast_convert_type(y, jnp.int32)
    i = jnp.int32(0x5F3759DF) - lax.shift_right_arithmetic(i, 1)
    r = lax.bitcast_convert_type(i, jnp.float32)
    half = 0.5 * y
    for _ in range(3):
        r = r * (1.5 - half * r * r)
    return r


def _emb_ln_body(x_hbm, ids_hbm, tab_hbm, g_hbm, b_hbm, out_hbm,
                 idxall_v, x0, x1, t0, t1, o0, o1, g_v, b_v, stage_v,
                 sg0, sg1, sx0, sx1, so0, so1):
    x_v = [x0, x1]
    t_v = [t0, t1]
    o_v = [o0, o1]
    sem_g = [sg0, sg1]
    sem_x = [sx0, sx1]
    sem_o = [so0, so1]

    wid = lax.axis_index("s") * NC + lax.axis_index("c")
    base_row = wid * ROWS_PER_W

    pltpu.sync_copy(g_hbm, g_v)
    pltpu.sync_copy(b_hbm, b_v)
    pltpu.sync_copy(ids_hbm.at[pl.ds(base_row, ROWS_PER_W)], idxall_v)

    zeros = jnp.zeros((LANES,), jnp.float32)

    def start_loads(c, bslot):
        row0 = base_row + c * CH
        pltpu.async_copy(tab_hbm.at[idxall_v.at[pl.ds(c * CH, CH)]],
                         t_v[bslot], sem_g[bslot])
        pltpu.async_copy(x_hbm.at[pl.ds(row0, CH)], x_v[bslot], sem_x[bslot])

    lane_iota = jnp.arange(LANES, dtype=jnp.int32)
    col_base = lane_iota * (2 * LANES)

    def pass1(bslot):
        xb, tb, ob = x_v[bslot], t_v[bslot], o_v[bslot]

        def row_stats(r, _):
            def acc(j, carry):
                accs = list(carry)
                for k in range(NACC):
                    sl = pl.ds((j + k) * LANES, LANES)
                    v = xb[r, sl] + tb[r, sl]
                    ob[r, sl] = v
                    accs[k] = accs[k] + v
                    accs[NACC + k] = accs[NACC + k] + v * v
                return tuple(accs)

            accs = plsc.parallel_loop(
                0, VPR, step=NACC, unroll=2,
                carry=(zeros,) * (2 * NACC))(acc)
            s = (accs[0] + accs[1]) + (accs[2] + accs[3])
            q = (accs[4] + accs[5]) + (accs[6] + accs[7])
            stage_v[pl.ds(r * 2 * LANES, LANES)] = s
            stage_v[pl.ds(r * 2 * LANES + LANES, LANES)] = q
            return 0

        lax.fori_loop(0, CH, row_stats, 0)

    def pass2(bslot):
        ob = o_v[bslot]

        srow = zeros
        qrow = zeros
        for l in range(LANES):
            srow = srow + plsc.load_gather(stage_v, [col_base + l])
            qrow = qrow + plsc.load_gather(stage_v, [col_base + LANES + l])
        mean_v = srow * (1.0 / HID)
        var_v = qrow * (1.0 / HID) - mean_v * mean_v
        rstd_v = _rsqrt(var_v + EPS)
        stage_v[pl.ds(STATS_OFF, LANES)] = mean_v
        stage_v[pl.ds(STATS_OFF + LANES, LANES)] = rstd_v

        bm = []
        br = []
        for r in range(CH):
            splat = jnp.full((LANES,), r, dtype=jnp.int32)
            bm.append(plsc.load_gather(stage_v, [STATS_OFF + splat]))
            br.append(plsc.load_gather(stage_v, [STATS_OFF + LANES + splat]))

        def col(j):
            sl = pl.ds(j * LANES, LANES)
            g = g_v[sl]
            b = b_v[sl]
            for r in range(CH):
                v = ob[r, sl]
                ob[r, sl] = (v - bm[r]) * br[r] * g + b

        plsc.parallel_loop(0, VPR, step=1, unroll=1)(col)

    for b in range(NBUF):
        start_loads(b, b)

    def body(c0, _):
        for b in range(NBUF):
            c = c0 + b
            row0 = base_row + c * CH
            pltpu.make_async_copy(
                tab_hbm.at[idxall_v.at[pl.ds(c * CH, CH)]], t_v[b],
                sem_g[b]).wait()
            pltpu.make_async_copy(x_hbm.at[pl.ds(row0, CH)], x_v[b],
                                  sem_x[b]).wait()

            @pl.when(c >= NBUF)
            def _():
                prev0 = base_row + (c - NBUF) * CH
                pltpu.make_async_copy(o_v[b], out_hbm.at[pl.ds(prev0, CH)],
                                      sem_o[b]).wait()

            pass1(b)

            @pl.when(c + NBUF < NCHUNKS)
            def _():
                start_loads(c + NBUF, b)

            pass2(b)
            pltpu.async_copy(o_v[b], out_hbm.at[pl.ds(row0, CH)], sem_o[b])

        return 0

    lax.fori_loop(0, NCHUNKS // NBUF, lambda i, _: body(i * NBUF, _), 0)

    for b in range(NBUF):
        c = NCHUNKS - NBUF + b
        row0 = base_row + c * CH
        pltpu.make_async_copy(o_v[b], out_hbm.at[pl.ds(row0, CH)],
                              sem_o[b]).wait()


def _tc_body(ids_ref, x_ref, tab_ref, g_ref, b_ref, o_ref):
    ids = ids_ref[0, 0, :]
    onehot = (
        jax.lax.broadcasted_iota(jnp.int32, (RB, MAXP), 1)
        == ids.reshape(RB, 1)
    ).astype(jnp.bfloat16)
    pos = jnp.dot(onehot, tab_ref[...],
                  preferred_element_type=jnp.float32)
    emb = x_ref[...] + pos
    mean = jnp.mean(emb, axis=-1, keepdims=True)
    var = jnp.mean(emb * emb, axis=-1, keepdims=True) - mean * mean
    normed = (emb - mean) * jax.lax.rsqrt(var + EPS)
    o_ref[...] = normed * g_ref[...] + b_ref[...]


def _tc_call(x_tc, ids_tc, tab_bf, gamma, beta):
    return pl.pallas_call(
        _tc_body,
        grid=(NB_TC,),
        in_specs=[
            pl.BlockSpec((1, 1, RB), lambda i: (i, 0, 0)),
            pl.BlockSpec((RB, HID), lambda i: (i, 0)),
            pl.BlockSpec((MAXP, HID), lambda i: (0, 0)),
            pl.BlockSpec((1, HID), lambda i: (0, 0)),
            pl.BlockSpec((1, HID), lambda i: (0, 0)),
        ],
        out_specs=pl.BlockSpec((RB, HID), lambda i: (i, 0)),
        out_shape=jax.ShapeDtypeStruct((N_TC, HID), jnp.float32),
    )(ids_tc.reshape(NB_TC, 1, RB), x_tc, tab_bf,
      gamma.reshape(1, HID), beta.reshape(1, HID))


@jax.jit
def _emb_ln(x, ids, tab, gamma, beta):
    mesh = plsc.VectorSubcoreMesh(core_axis_name="c", subcore_axis_name="s")
    out_sc = pl.kernel(
        _emb_ln_body,
        out_type=jax.ShapeDtypeStruct((N_SC, HID), jnp.float32),
        mesh=mesh,
        compiler_params=pltpu.CompilerParams(needs_layout_passes=False),
        scratch_types=[
            pltpu.VMEM((ROWS_PER_W,), jnp.int32),
            pltpu.VMEM((CH, HID), jnp.float32),
            pltpu.VMEM((CH, HID), jnp.float32),
            pltpu.VMEM((CH, HID), jnp.float32),
            pltpu.VMEM((CH, HID), jnp.float32),
            pltpu.VMEM((CH, HID), jnp.float32),
            pltpu.VMEM((CH, HID), jnp.float32),
            pltpu.VMEM((HID,), jnp.float32),
            pltpu.VMEM((HID,), jnp.float32),
            pltpu.VMEM(((CH * 2 + 2) * LANES,), jnp.float32),
            pltpu.SemaphoreType.DMA,
            pltpu.SemaphoreType.DMA,
            pltpu.SemaphoreType.DMA,
            pltpu.SemaphoreType.DMA,
            pltpu.SemaphoreType.DMA,
            pltpu.SemaphoreType.DMA,
        ],
    )(x[:N_SC], ids[:N_SC], tab, gamma, beta)
    out_tc = _tc_call(x[N_SC:], ids[N_SC:], tab.astype(jnp.bfloat16),
                      gamma, beta)
    return jnp.concatenate([out_sc, out_tc], axis=0)


def kernel(input_embeds, position_ids, pos_table, gamma, beta):
    x = input_embeds.reshape(N, HID)
    ids = position_ids.reshape(N)
    out = _emb_ln(x, ids, pos_table, gamma, beta)
    return out.reshape(B, S, HID)


# speedup vs baseline: 1.6166x; 1.6166x over previous
import functools

import jax
import jax.numpy as jnp
from jax import lax
from jax.experimental import pallas as pl
from jax.experimental.pallas import tpu as pltpu
from jax.experimental.pallas import tpu_sc as plsc

B, S, HID, MAXP = 4, 4096, 1024, 4096
EPS = 1e-12
N = B * S

LANES = 16
VPR = HID // LANES
CH = 16
NC, NS = 2, 16
NW = NC * NS
ROWS_PER_W = N // NW
NCHUNKS = ROWS_PER_W // CH
NBUF = 2
NACC = 4
STATS_OFF = CH * 2 * LANES


def _rsqrt(y):
    i = lax.bitcast_convert_type(y, jnp.int32)
    i = jnp.int32(0x5F3759DF) - lax.shift_right_arithmetic(i, 1)
    r = lax.bitcast_convert_type(i, jnp.float32)
    half = 0.5 * y
    for _ in range(3):
        r = r * (1.5 - half * r * r)
    return r


def _emb_ln_body(x_hbm, ids_hbm, tab_hbm, g_hbm, b_hbm, out_hbm,
                 idxall_v, x0, x1, t0, t1, o0, o1, g_v, b_v, stage_v,
                 sg0, sg1, sx0, sx1, so0, so1):
    x_v = [x0, x1]
    t_v = [t0, t1]
    o_v = [o0, o1]
    sem_g = [sg0, sg1]
    sem_x = [sx0, sx1]
    sem_o = [so0, so1]

    wid = lax.axis_index("s") * NC + lax.axis_index("c")
    base_row = wid * ROWS_PER_W

    pltpu.sync_copy(g_hbm, g_v)
    pltpu.sync_copy(b_hbm, b_v)
    pltpu.sync_copy(ids_hbm.at[pl.ds(base_row, ROWS_PER_W)], idxall_v)

    zeros = jnp.zeros((LANES,), jnp.float32)

    def start_loads(c, bslot):
        row0 = base_row + c * CH
        pltpu.async_copy(tab_hbm.at[idxall_v.at[pl.ds(c * CH, CH)]],
                         t_v[bslot], sem_g[bslot])
        pltpu.async_copy(x_hbm.at[pl.ds(row0, CH)], x_v[bslot], sem_x[bslot])

    lane_iota = jnp.arange(LANES, dtype=jnp.int32)
    col_base = lane_iota * (2 * LANES)

    def pass1(bslot):
        xb, tb, ob = x_v[bslot], t_v[bslot], o_v[bslot]

        def row_stats(r, _):
            def acc(j, carry):
                accs = list(carry)
                for k in range(NACC):
                    sl = pl.ds((j + k) * LANES, LANES)
                    v = xb[r, sl] + tb[r, sl]
                    ob[r, sl] = v
                    accs[k] = accs[k] + v
                    accs[NACC + k] = accs[NACC + k] + v * v
                return tuple(accs)

            accs = plsc.parallel_loop(
                0, VPR, step=NACC, unroll=2,
                carry=(zeros,) * (2 * NACC))(acc)
            s = (accs[0] + accs[1]) + (accs[2] + accs[3])
            q = (accs[4] + accs[5]) + (accs[6] + accs[7])
            stage_v[pl.ds(r * 2 * LANES, LANES)] = s
            stage_v[pl.ds(r * 2 * LANES + LANES, LANES)] = q
            return 0

        lax.fori_loop(0, CH, row_stats, 0)

    def pass2(bslot):
        ob = o_v[bslot]

        srow = zeros
        qrow = zeros
        for l in range(LANES):
            srow = srow + plsc.load_gather(stage_v, [col_base + l])
            qrow = qrow + plsc.load_gather(stage_v, [col_base + LANES + l])
        mean_v = srow * (1.0 / HID)
        var_v = qrow * (1.0 / HID) - mean_v * mean_v
        rstd_v = _rsqrt(var_v + EPS)
        stage_v[pl.ds(STATS_OFF, LANES)] = mean_v
        stage_v[pl.ds(STATS_OFF + LANES, LANES)] = rstd_v

        bm = []
        br = []
        for r in range(CH):
            splat = jnp.full((LANES,), r, dtype=jnp.int32)
            bm.append(plsc.load_gather(stage_v, [STATS_OFF + splat]))
            br.append(plsc.load_gather(stage_v, [STATS_OFF + LANES + splat]))

        def col(j):
            sl = pl.ds(j * LANES, LANES)
            g = g_v[sl]
            b = b_v[sl]
            for r in range(CH):
                v = ob[r, sl]
                ob[r, sl] = (v - bm[r]) * br[r] * g + b

        plsc.parallel_loop(0, VPR, step=1, unroll=1)(col)

    for b in range(NBUF):
        start_loads(b, b)

    def body(c0, _):
        for b in range(NBUF):
            c = c0 + b
            row0 = base_row + c * CH
            pltpu.make_async_copy(
                tab_hbm.at[idxall_v.at[pl.ds(c * CH, CH)]], t_v[b],
                sem_g[b]).wait()
            pltpu.make_async_copy(x_hbm.at[pl.ds(row0, CH)], x_v[b],
                                  sem_x[b]).wait()

            @pl.when(c >= NBUF)
            def _():
                prev0 = base_row + (c - NBUF) * CH
                pltpu.make_async_copy(o_v[b], out_hbm.at[pl.ds(prev0, CH)],
                                      sem_o[b]).wait()

            pass1(b)

            @pl.when(c + NBUF < NCHUNKS)
            def _():
                start_loads(c + NBUF, b)

            pass2(b)
            pltpu.async_copy(o_v[b], out_hbm.at[pl.ds(row0, CH)], sem_o[b])

        return 0

    lax.fori_loop(0, NCHUNKS // NBUF, lambda i, _: body(i * NBUF, _), 0)

    for b in range(NBUF):
        c = NCHUNKS - NBUF + b
        row0 = base_row + c * CH
        pltpu.make_async_copy(o_v[b], out_hbm.at[pl.ds(row0, CH)],
                              sem_o[b]).wait()


@jax.jit
def _emb_ln(x, ids, tab, gamma, beta):
    mesh = plsc.VectorSubcoreMesh(core_axis_name="c", subcore_axis_name="s")
    return pl.kernel(
        _emb_ln_body,
        out_type=jax.ShapeDtypeStruct((N, HID), jnp.float32),
        mesh=mesh,
        compiler_params=pltpu.CompilerParams(needs_layout_passes=False),
        scratch_types=[
            pltpu.VMEM((ROWS_PER_W,), jnp.int32),
            pltpu.VMEM((CH, HID), jnp.float32),
            pltpu.VMEM((CH, HID), jnp.float32),
            pltpu.VMEM((CH, HID), jnp.float32),
            pltpu.VMEM((CH, HID), jnp.float32),
            pltpu.VMEM((CH, HID), jnp.float32),
            pltpu.VMEM((CH, HID), jnp.float32),
            pltpu.VMEM((HID,), jnp.float32),
            pltpu.VMEM((HID,), jnp.float32),
            pltpu.VMEM(((CH * 2 + 2) * LANES,), jnp.float32),
            pltpu.SemaphoreType.DMA,
            pltpu.SemaphoreType.DMA,
            pltpu.SemaphoreType.DMA,
            pltpu.SemaphoreType.DMA,
            pltpu.SemaphoreType.DMA,
            pltpu.SemaphoreType.DMA,
        ],
    )(x, ids, tab, gamma, beta)


def kernel(input_embeds, position_ids, pos_table, gamma, beta):
    x = input_embeds.reshape(N, HID)
    ids = position_ids.reshape(N)
    out = _emb_ln(x, ids, pos_table, gamma, beta)
    return out.reshape(B, S, HID)
